# baseline (speedup 1.0000x reference)
import jax
import jax.numpy as jnp
from jax import lax
from jax.experimental import pallas as pl
from jax.experimental.pallas import tpu as pltpu
from jax.experimental.pallas import tpu_sc as plsc

DIM = 768
E = 16
N = 2048
BM = 128
BM_SHIFT = 7
NT = N // BM
MAX_VISITS = NT + E - 1

NW = 32
ROWS_W = N // NW


def _router_body(x_ref, gw_ref, dest_ref, meta_ref, xsc_ref):
    x = x_ref[...]
    gw = gw_ref[...]
    logits = lax.dot_general(x, gw, (((1,), (1,)), ((), ())),
                             preferred_element_type=jnp.float32)
    m = jnp.max(logits, axis=1, keepdims=True)
    eids = lax.broadcasted_iota(jnp.int32, (N, E), 1)
    expert = jnp.min(jnp.where(logits == m, eids, E), axis=1, keepdims=True)
    onehot = (eids == expert).astype(jnp.bfloat16)
    ones_col = jnp.ones((N, 8), dtype=jnp.bfloat16)
    counts_c = lax.dot_general(onehot, ones_col, (((0,), (0,)), ((), ())),
                               preferred_element_type=jnp.float32)
    bi = lax.broadcasted_iota(jnp.int32, (N, N), 0)
    bj = lax.broadcasted_iota(jnp.int32, (N, N), 1)
    tri = (bi > bj).astype(jnp.bfloat16)
    csum = lax.dot_general(tri, onehot, (((1,), (0,)), ((), ())),
                           preferred_element_type=jnp.float32)
    oh_f = onehot.astype(jnp.float32)
    rank = jnp.sum(csum * oh_f, axis=1, keepdims=True)
    si = lax.broadcasted_iota(jnp.int32, (E, E), 0)
    sj = lax.broadcasted_iota(jnp.int32, (E, E), 1)
    tri_le = (si >= sj).astype(jnp.float32)
    cum_c = lax.dot_general(tri_le, counts_c, (((1,), (0,)), ((), ())),
                            preferred_element_type=jnp.float32)
    counts_i = counts_c[:, 0:1].astype(jnp.int32)
    ends_i = cum_c[:, 0:1].astype(jnp.int32)
    starts_i = ends_i - counts_i
    counts_lane = lax.dot_general(
        jnp.ones((8, N), jnp.bfloat16), onehot, (((1,), (0,)), ((), ())),
        preferred_element_type=jnp.float32)[0:1]
    tri_lt = (si < sj).astype(jnp.float32)
    starts_lane = lax.dot_general(counts_lane, tri_lt,
                                  (((1,), (0,)), ((), ())),
                                  preferred_element_type=jnp.float32)
    starts_row = jnp.sum(jnp.where(eids == expert, starts_lane, 0.0),
                         axis=1, keepdims=True)
    dest_ref[...] = (rank + starts_row).astype(jnp.int32)
    xsc_ref[...] = x * jax.nn.sigmoid(m)

    t0 = lax.shift_right_logical(starts_i, BM_SHIFT)
    t1 = lax.shift_right_logical(ends_i + (BM - 1), BM_SHIFT)
    ntiles = jnp.where(counts_i > 0, t1 - t0, 0)
    cum_nt = lax.dot_general(
        tri_le, ntiles.astype(jnp.float32), (((1,), (0,)), ((), ())),
        preferred_element_type=jnp.float32).astype(jnp.int32)
    offs = cum_nt - ntiles
    total = jnp.max(cum_nt)
    vi = lax.broadcasted_iota(jnp.int32, (1, MAX_VISITS), 1)
    e_of = jnp.minimum(
        jnp.sum((cum_nt <= vi).astype(jnp.int32), axis=0, keepdims=True),
        E - 1)
    sel = (lax.broadcasted_iota(jnp.int32, (E, MAX_VISITS), 0) == e_of)
    def pick(col):
        return jnp.sum(jnp.where(sel, col, 0), axis=0, keepdims=True)
    t0_s = pick(t0)
    offs_s = pick(offs)
    starts_s = pick(starts_i)
    ends_s = pick(ends_i)
    valid = vi < total
    tile = jnp.where(valid, t0_s + vi - offs_s, NT - 1)
    lo = jnp.where(valid, jnp.maximum(starts_s, tile * BM), 0)
    hi = jnp.where(valid, jnp.minimum(ends_s, (tile + 1) * BM), 0)
    first = (lo == tile * BM).astype(jnp.int32)
    expv = jnp.where(valid, e_of, 0)
    meta_ref[...] = jnp.zeros((8, 128), jnp.int32)
    meta_ref[0:1, 0:MAX_VISITS] = tile
    meta_ref[1:2, 0:MAX_VISITS] = expv
    meta_ref[2:3, 0:MAX_VISITS] = lo
    meta_ref[3:4, 0:MAX_VISITS] = hi
    meta_ref[4:5, 0:MAX_VISITS] = first


def _router(x2d, gate_w):
    return pl.pallas_call(
        _router_body,
        out_shape=(
            jax.ShapeDtypeStruct((N, 1), jnp.int32),
            jax.ShapeDtypeStruct((8, 128), jnp.int32),
            jax.ShapeDtypeStruct((N, DIM), jnp.float32),
        ),
    )(x2d, gate_w)


def _dispatch_body(xw_hbm, dest_hbm, xs_hbm, idx_v, rows_v, sem):
    wid = lax.axis_index("s") * 2 + lax.axis_index("c")
    base = wid * ROWS_W
    pltpu.sync_copy(dest_hbm.at[pl.ds(base, ROWS_W)], idx_v)
    pltpu.sync_copy(xw_hbm.at[pl.ds(base, ROWS_W)], rows_v)
    pltpu.async_copy(rows_v, xs_hbm.at[idx_v], sem).wait()


def _dispatch(xw, dest):
    mesh = plsc.VectorSubcoreMesh(core_axis_name="c", subcore_axis_name="s")
    return pl.kernel(
        _dispatch_body,
        out_type=jax.ShapeDtypeStruct((N, DIM), jnp.float32),
        mesh=mesh,
        scratch_types=[
            pltpu.VMEM((ROWS_W,), jnp.int32),
            pltpu.VMEM((ROWS_W, DIM), jnp.float32),
            pltpu.SemaphoreType.DMA,
        ],
    )(xw, dest)


def _gmm_body(meta_r, xs_ref, w1_ref, w3_ref, w2_ref, out_ref):
    i = pl.program_id(0)
    lo = meta_r[2, i]
    hi = meta_r[3, i]
    tile = meta_r[0, i]
    rows = tile * BM + lax.broadcasted_iota(jnp.int32, (BM, 1), 0)
    mask = (rows >= lo) & (rows < hi)
    xb = jnp.where(mask, xs_ref[...], 0.0).astype(jnp.bfloat16)
    dn = (((1,), (1,)), ((), ()))
    z1 = lax.dot_general(xb, w1_ref[0].astype(jnp.bfloat16), dn,
                         preferred_element_type=jnp.float32).astype(jnp.bfloat16)
    z3 = lax.dot_general(xb, w3_ref[0].astype(jnp.bfloat16), dn,
                         preferred_element_type=jnp.float32).astype(jnp.bfloat16)
    h = (z1 * jax.nn.sigmoid(z1)) * z3
    o = lax.dot_general(h, w2_ref[0].astype(jnp.bfloat16), dn,
                        preferred_element_type=jnp.float32)
    o = o.astype(jnp.bfloat16).astype(jnp.float32)

    @pl.when(meta_r[4, i] == 1)
    def _():
        out_ref[...] = o

    @pl.when(meta_r[4, i] == 0)
    def _():
        out_ref[...] = out_ref[...] + o


def _gmm(xs, w1b, w3b, w2b, meta):
    grid_spec = pltpu.PrefetchScalarGridSpec(
        num_scalar_prefetch=1,
        grid=(MAX_VISITS,),
        in_specs=[
            pl.BlockSpec((BM, DIM), lambda i, m: (m[0, i], 0)),
            pl.BlockSpec((1, DIM, DIM), lambda i, m: (m[1, i], 0, 0)),
            pl.BlockSpec((1, DIM, DIM), lambda i, m: (m[1, i], 0, 0)),
            pl.BlockSpec((1, DIM, DIM), lambda i, m: (m[1, i], 0, 0)),
        ],
        out_specs=pl.BlockSpec((BM, DIM), lambda i, m: (m[0, i], 0)),
    )
    return pl.pallas_call(
        _gmm_body,
        grid_spec=grid_spec,
        out_shape=jax.ShapeDtypeStruct((N, DIM), jnp.float32),
        compiler_params=pltpu.CompilerParams(
            dimension_semantics=("arbitrary",)),
    )(meta, xs, w1b, w3b, w2b)


def _combine_body(y_hbm, dest_hbm, out_hbm, idx_v, rows_v, sem):
    wid = lax.axis_index("s") * 2 + lax.axis_index("c")
    base = wid * ROWS_W
    pltpu.sync_copy(dest_hbm.at[pl.ds(base, ROWS_W)], idx_v)
    pltpu.async_copy(y_hbm.at[idx_v], rows_v, sem).wait()
    pltpu.sync_copy(rows_v, out_hbm.at[pl.ds(base, ROWS_W)])


def _combine(yw, dest):
    mesh = plsc.VectorSubcoreMesh(core_axis_name="c", subcore_axis_name="s")
    return pl.kernel(
        _combine_body,
        out_type=jax.ShapeDtypeStruct((N, DIM), jnp.float32),
        mesh=mesh,
        scratch_types=[
            pltpu.VMEM((ROWS_W,), jnp.int32),
            pltpu.VMEM((ROWS_W, DIM), jnp.float32),
            pltpu.SemaphoreType.DMA,
        ],
    )(yw, dest)


@jax.jit
def kernel(x, gate_w, w1, w2, w3):
    bs, slen, dim = x.shape
    x2d = x.reshape(N, DIM)

    dest2d, meta, xsc = _router(x2d, gate_w)
    dest = dest2d.reshape(N)

    xs = _dispatch(xsc, dest)
    y = _gmm(xs, w1, w3, w2, meta)

    out = _combine(y, dest)
    return out.reshape(bs, slen, dim)


# speedup vs baseline: 1.2348x; 1.2348x over previous
import jax
import jax.numpy as jnp
from jax import lax
from jax.experimental import pallas as pl
from jax.experimental.pallas import tpu as pltpu
from jax.experimental.pallas import tpu_sc as plsc

DIM = 768
DIMH = DIM // 2
E = 16
N = 2048
BM = 256
BM_SHIFT = 8
NT = N // BM
MAX_VISITS = NT + E - 1

NW = 32
ROWS_W = N // NW


def _router_body(x_ref, gw_ref, dest_ref, meta_ref, xsc_ref):
    x = x_ref[...]
    gw = gw_ref[...]
    logits = lax.dot_general(x, gw, (((1,), (1,)), ((), ())),
                             preferred_element_type=jnp.float32)
    m = jnp.max(logits, axis=1, keepdims=True)
    eids = lax.broadcasted_iota(jnp.int32, (N, E), 1)
    expert = jnp.min(jnp.where(logits == m, eids, E), axis=1, keepdims=True)
    onehot = (eids == expert).astype(jnp.bfloat16)
    ones_col = jnp.ones((N, 8), dtype=jnp.bfloat16)
    counts_c = lax.dot_general(onehot, ones_col, (((0,), (0,)), ((), ())),
                               preferred_element_type=jnp.float32)
    bi = lax.broadcasted_iota(jnp.int32, (N, N), 0)
    bj = lax.broadcasted_iota(jnp.int32, (N, N), 1)
    tri = (bi > bj).astype(jnp.bfloat16)
    csum = lax.dot_general(tri, onehot, (((1,), (0,)), ((), ())),
                           preferred_element_type=jnp.float32)
    oh_f = onehot.astype(jnp.float32)
    rank = jnp.sum(csum * oh_f, axis=1, keepdims=True)
    si = lax.broadcasted_iota(jnp.int32, (E, E), 0)
    sj = lax.broadcasted_iota(jnp.int32, (E, E), 1)
    tri_le = (si >= sj).astype(jnp.float32)
    cum_c = lax.dot_general(tri_le, counts_c, (((1,), (0,)), ((), ())),
                            preferred_element_type=jnp.float32)
    counts_i = counts_c[:, 0:1].astype(jnp.int32)
    ends_i = cum_c[:, 0:1].astype(jnp.int32)
    starts_i = ends_i - counts_i
    counts_lane = lax.dot_general(
        jnp.ones((8, N), jnp.bfloat16), onehot, (((1,), (0,)), ((), ())),
        preferred_element_type=jnp.float32)[0:1]
    tri_lt = (si < sj).astype(jnp.float32)
    starts_lane = lax.dot_general(counts_lane, tri_lt,
                                  (((1,), (0,)), ((), ())),
                                  preferred_element_type=jnp.float32)
    starts_row = jnp.sum(jnp.where(eids == expert, starts_lane, 0.0),
                         axis=1, keepdims=True)
    dest_ref[...] = (rank + starts_row).astype(jnp.int32)
    zb = (x * jax.nn.sigmoid(m)).astype(jnp.bfloat16)
    lo32 = lax.convert_element_type(
        lax.bitcast_convert_type(zb[:, :DIMH], jnp.uint16), jnp.uint32)
    hi32 = lax.convert_element_type(
        lax.bitcast_convert_type(zb[:, DIMH:], jnp.uint16), jnp.uint32)
    xsc_ref[...] = lax.bitcast_convert_type(
        jnp.bitwise_or(lax.shift_left(hi32, jnp.uint32(16)), lo32), jnp.int32)

    t0 = lax.shift_right_logical(starts_i, BM_SHIFT)
    t1 = lax.shift_right_logical(ends_i + (BM - 1), BM_SHIFT)
    ntiles = jnp.where(counts_i > 0, t1 - t0, 0)
    cum_nt = lax.dot_general(
        tri_le, ntiles.astype(jnp.float32), (((1,), (0,)), ((), ())),
        preferred_element_type=jnp.float32).astype(jnp.int32)
    offs = cum_nt - ntiles
    total = jnp.max(cum_nt)
    vi = lax.broadcasted_iota(jnp.int32, (1, MAX_VISITS), 1)
    e_of = jnp.minimum(
        jnp.sum((cum_nt <= vi).astype(jnp.int32), axis=0, keepdims=True),
        E - 1)
    sel = (lax.broadcasted_iota(jnp.int32, (E, MAX_VISITS), 0) == e_of)
    def pick(col):
        return jnp.sum(jnp.where(sel, col, 0), axis=0, keepdims=True)
    t0_s = pick(t0)
    offs_s = pick(offs)
    starts_s = pick(starts_i)
    ends_s = pick(ends_i)
    valid = vi < total
    tile = jnp.where(valid, t0_s + vi - offs_s, NT - 1)
    lo = jnp.where(valid, jnp.maximum(starts_s, tile * BM), 0)
    hi = jnp.where(valid, jnp.minimum(ends_s, (tile + 1) * BM), 0)
    first = (lo == tile * BM).astype(jnp.int32)
    expv = jnp.where(valid, e_of, 0)
    meta_ref[...] = jnp.zeros((8, 128), jnp.int32)
    meta_ref[0:1, 0:MAX_VISITS] = tile
    meta_ref[1:2, 0:MAX_VISITS] = expv
    meta_ref[2:3, 0:MAX_VISITS] = lo
    meta_ref[3:4, 0:MAX_VISITS] = hi
    meta_ref[4:5, 0:MAX_VISITS] = first


def _router(x2d, gate_w):
    return pl.pallas_call(
        _router_body,
        out_shape=(
            jax.ShapeDtypeStruct((N, 1), jnp.int32),
            jax.ShapeDtypeStruct((8, 128), jnp.int32),
            jax.ShapeDtypeStruct((N, DIMH), jnp.int32),
        ),
    )(x2d, gate_w)


def _dispatch_body(xw_hbm, dest_hbm, xs_hbm, idx_v, rows_v, sem):
    wid = lax.axis_index("s") * 2 + lax.axis_index("c")
    base = wid * ROWS_W
    pltpu.sync_copy(dest_hbm.at[pl.ds(base, ROWS_W)], idx_v)
    pltpu.sync_copy(xw_hbm.at[pl.ds(base, ROWS_W)], rows_v)
    pltpu.async_copy(rows_v, xs_hbm.at[idx_v], sem).wait()


def _dispatch(xw, dest):
    mesh = plsc.VectorSubcoreMesh(core_axis_name="c", subcore_axis_name="s")
    return pl.kernel(
        _dispatch_body,
        out_type=jax.ShapeDtypeStruct((N, DIMH), jnp.int32),
        mesh=mesh,
        scratch_types=[
            pltpu.VMEM((ROWS_W,), jnp.int32),
            pltpu.VMEM((ROWS_W, DIMH), jnp.int32),
            pltpu.SemaphoreType.DMA,
        ],
    )(xw, dest)


def _gmm_body(meta_r, xs_ref, w1_ref, w3_ref, w2_ref, out_ref):
    i = pl.program_id(0)
    lo = meta_r[2, i]
    hi = meta_r[3, i]
    tile = meta_r[0, i]
    rows = tile * BM + lax.broadcasted_iota(jnp.int32, (BM, 1), 0)
    mask = (rows >= lo) & (rows < hi)
    wu = lax.bitcast_convert_type(xs_ref[...], jnp.uint32)
    lo_bf = lax.bitcast_convert_type(
        lax.convert_element_type(
            jnp.bitwise_and(wu, jnp.uint32(0xFFFF)), jnp.uint16),
        jnp.bfloat16)
    hi_bf = lax.bitcast_convert_type(
        lax.convert_element_type(
            lax.shift_right_logical(wu, jnp.uint32(16)), jnp.uint16),
        jnp.bfloat16)
    xb = jnp.where(mask, jnp.concatenate([lo_bf, hi_bf], axis=1),
                   jnp.bfloat16(0))
    dn = (((1,), (1,)), ((), ()))
    z1 = lax.dot_general(xb, w1_ref[0].astype(jnp.bfloat16), dn,
                         preferred_element_type=jnp.float32).astype(jnp.bfloat16)
    z3 = lax.dot_general(xb, w3_ref[0].astype(jnp.bfloat16), dn,
                         preferred_element_type=jnp.float32).astype(jnp.bfloat16)
    h = (z1 * jax.nn.sigmoid(z1)) * z3
    o = lax.dot_general(h, w2_ref[0].astype(jnp.bfloat16), dn,
                        preferred_element_type=jnp.float32)
    o = o.astype(jnp.bfloat16).astype(jnp.float32)

    @pl.when(meta_r[4, i] == 1)
    def _():
        out_ref[...] = o

    @pl.when(meta_r[4, i] == 0)
    def _():
        out_ref[...] = out_ref[...] + o


def _gmm(xs, w1b, w3b, w2b, meta):
    grid_spec = pltpu.PrefetchScalarGridSpec(
        num_scalar_prefetch=1,
        grid=(MAX_VISITS,),
        in_specs=[
            pl.BlockSpec((BM, DIMH), lambda i, m: (m[0, i], 0)),
            pl.BlockSpec((1, DIM, DIM), lambda i, m: (m[1, i], 0, 0)),
            pl.BlockSpec((1, DIM, DIM), lambda i, m: (m[1, i], 0, 0)),
            pl.BlockSpec((1, DIM, DIM), lambda i, m: (m[1, i], 0, 0)),
        ],
        out_specs=pl.BlockSpec((BM, DIM), lambda i, m: (m[0, i], 0)),
    )
    return pl.pallas_call(
        _gmm_body,
        grid_spec=grid_spec,
        out_shape=jax.ShapeDtypeStruct((N, DIM), jnp.float32),
        compiler_params=pltpu.CompilerParams(
            dimension_semantics=("arbitrary",)),
    )(meta, xs, w1b, w3b, w2b)


def _combine_body(y_hbm, dest_hbm, out_hbm, idx_v, rows_v, sem):
    wid = lax.axis_index("s") * 2 + lax.axis_index("c")
    base = wid * ROWS_W
    pltpu.sync_copy(dest_hbm.at[pl.ds(base, ROWS_W)], idx_v)
    pltpu.async_copy(y_hbm.at[idx_v], rows_v, sem).wait()
    pltpu.sync_copy(rows_v, out_hbm.at[pl.ds(base, ROWS_W)])


def _combine(yw, dest):
    mesh = plsc.VectorSubcoreMesh(core_axis_name="c", subcore_axis_name="s")
    return pl.kernel(
        _combine_body,
        out_type=jax.ShapeDtypeStruct((N, DIM), jnp.float32),
        mesh=mesh,
        scratch_types=[
            pltpu.VMEM((ROWS_W,), jnp.int32),
            pltpu.VMEM((ROWS_W, DIM), jnp.float32),
            pltpu.SemaphoreType.DMA,
        ],
    )(yw, dest)


@jax.jit
def kernel(x, gate_w, w1, w2, w3):
    bs, slen, dim = x.shape
    x2d = x.reshape(N, DIM)

    dest2d, meta, xsc = _router(x2d, gate_w)
    dest = dest2d.reshape(N)

    xs = _dispatch(xsc, dest)
    y = _gmm(xs, w1, w3, w2, meta)

    out = _combine(y, dest)
    return out.reshape(bs, slen, dim)


# speedup vs baseline: 1.2490x; 1.0115x over previous
import jax
import jax.numpy as jnp
from jax import lax
from jax.experimental import pallas as pl
from jax.experimental.pallas import tpu as pltpu
from jax.experimental.pallas import tpu_sc as plsc

DIM = 768
DIMH = DIM // 2
E = 16
N = 2048
BM = 256
BM_SHIFT = 8
NT = N // BM
MAX_VISITS = NT + E - 1

NW = 32
ROWS_W = N // NW


def _router_body(x_ref, gw_ref, dest_ref, meta_ref, xsc_ref):
    x = x_ref[...]
    gw = gw_ref[...]
    logits = lax.dot_general(x, gw, (((1,), (1,)), ((), ())),
                             preferred_element_type=jnp.float32)
    m = jnp.max(logits, axis=1, keepdims=True)
    eids = lax.broadcasted_iota(jnp.int32, (N, E), 1)
    expert = jnp.min(jnp.where(logits == m, eids, E), axis=1, keepdims=True)
    onehot = (eids == expert).astype(jnp.bfloat16)
    ones_col = jnp.ones((N, 8), dtype=jnp.bfloat16)
    counts_c = lax.dot_general(onehot, ones_col, (((0,), (0,)), ((), ())),
                               preferred_element_type=jnp.float32)
    NB = 256
    bi = lax.broadcasted_iota(jnp.int32, (NB, NB), 0)
    bj = lax.broadcasted_iota(jnp.int32, (NB, NB), 1)
    tri = (bi > bj).astype(jnp.bfloat16)
    oh_f = onehot.astype(jnp.float32)
    rank_blocks = []
    prefix = jnp.zeros((1, E), jnp.float32)
    for b in range(N // NB):
        oh_b = onehot[b * NB:(b + 1) * NB]
        csum_b = lax.dot_general(tri, oh_b, (((1,), (0,)), ((), ())),
                                 preferred_element_type=jnp.float32)
        rank_blocks.append(jnp.sum(
            (csum_b + prefix) * oh_f[b * NB:(b + 1) * NB],
            axis=1, keepdims=True))
        prefix = prefix + jnp.sum(oh_b.astype(jnp.float32), axis=0,
                                  keepdims=True)
    rank = jnp.concatenate(rank_blocks, axis=0)
    si = lax.broadcasted_iota(jnp.int32, (E, E), 0)
    sj = lax.broadcasted_iota(jnp.int32, (E, E), 1)
    tri_le = (si >= sj).astype(jnp.float32)
    cum_c = lax.dot_general(tri_le, counts_c, (((1,), (0,)), ((), ())),
                            preferred_element_type=jnp.float32)
    counts_i = counts_c[:, 0:1].astype(jnp.int32)
    ends_i = cum_c[:, 0:1].astype(jnp.int32)
    starts_i = ends_i - counts_i
    counts_lane = lax.dot_general(
        jnp.ones((8, N), jnp.bfloat16), onehot, (((1,), (0,)), ((), ())),
        preferred_element_type=jnp.float32)[0:1]
    tri_lt = (si < sj).astype(jnp.float32)
    starts_lane = lax.dot_general(counts_lane, tri_lt,
                                  (((1,), (0,)), ((), ())),
                                  preferred_element_type=jnp.float32)
    starts_row = jnp.sum(jnp.where(eids == expert, starts_lane, 0.0),
                         axis=1, keepdims=True)
    dest_ref[...] = (rank + starts_row).astype(jnp.int32)
    zb = (x * jax.nn.sigmoid(m)).astype(jnp.bfloat16)
    lo32 = lax.convert_element_type(
        lax.bitcast_convert_type(zb[:, :DIMH], jnp.uint16), jnp.uint32)
    hi32 = lax.convert_element_type(
        lax.bitcast_convert_type(zb[:, DIMH:], jnp.uint16), jnp.uint32)
    xsc_ref[...] = lax.bitcast_convert_type(
        jnp.bitwise_or(lax.shift_left(hi32, jnp.uint32(16)), lo32), jnp.int32)

    t0 = lax.shift_right_logical(starts_i, BM_SHIFT)
    t1 = lax.shift_right_logical(ends_i + (BM - 1), BM_SHIFT)
    ntiles = jnp.where(counts_i > 0, t1 - t0, 0)
    cum_nt = lax.dot_general(
        tri_le, ntiles.astype(jnp.float32), (((1,), (0,)), ((), ())),
        preferred_element_type=jnp.float32).astype(jnp.int32)
    offs = cum_nt - ntiles
    total = jnp.max(cum_nt)
    vi = lax.broadcasted_iota(jnp.int32, (1, MAX_VISITS), 1)
    e_of = jnp.minimum(
        jnp.sum((cum_nt <= vi).astype(jnp.int32), axis=0, keepdims=True),
        E - 1)
    sel = (lax.broadcasted_iota(jnp.int32, (E, MAX_VISITS), 0) == e_of)
    def pick(col):
        return jnp.sum(jnp.where(sel, col, 0), axis=0, keepdims=True)
    t0_s = pick(t0)
    offs_s = pick(offs)
    starts_s = pick(starts_i)
    ends_s = pick(ends_i)
    valid = vi < total
    tile = jnp.where(valid, t0_s + vi - offs_s, NT - 1)
    lo = jnp.where(valid, jnp.maximum(starts_s, tile * BM), 0)
    hi = jnp.where(valid, jnp.minimum(ends_s, (tile + 1) * BM), 0)
    first = (lo == tile * BM).astype(jnp.int32)
    expv = jnp.where(valid, e_of, 0)
    meta_ref[...] = jnp.zeros((8, 128), jnp.int32)
    meta_ref[0:1, 0:MAX_VISITS] = tile
    meta_ref[1:2, 0:MAX_VISITS] = expv
    meta_ref[2:3, 0:MAX_VISITS] = lo
    meta_ref[3:4, 0:MAX_VISITS] = hi
    meta_ref[4:5, 0:MAX_VISITS] = first


def _router(x2d, gate_w):
    return pl.pallas_call(
        _router_body,
        out_shape=(
            jax.ShapeDtypeStruct((N, 1), jnp.int32),
            jax.ShapeDtypeStruct((8, 128), jnp.int32),
            jax.ShapeDtypeStruct((N, DIMH), jnp.int32),
        ),
    )(x2d, gate_w)


def _dispatch_body(xw_hbm, dest_hbm, xs_hbm, idx_v, rows_v, sem):
    wid = lax.axis_index("s") * 2 + lax.axis_index("c")
    base = wid * ROWS_W
    pltpu.sync_copy(dest_hbm.at[pl.ds(base, ROWS_W)], idx_v)
    pltpu.sync_copy(xw_hbm.at[pl.ds(base, ROWS_W)], rows_v)
    pltpu.async_copy(rows_v, xs_hbm.at[idx_v], sem).wait()


def _dispatch(xw, dest):
    mesh = plsc.VectorSubcoreMesh(core_axis_name="c", subcore_axis_name="s")
    return pl.kernel(
        _dispatch_body,
        out_type=jax.ShapeDtypeStruct((N, DIMH), jnp.int32),
        mesh=mesh,
        scratch_types=[
            pltpu.VMEM((ROWS_W,), jnp.int32),
            pltpu.VMEM((ROWS_W, DIMH), jnp.int32),
            pltpu.SemaphoreType.DMA,
        ],
    )(xw, dest)


def _gmm_body(meta_r, xs_ref, w1_ref, w3_ref, w2_ref, out_ref):
    i = pl.program_id(0)
    lo = meta_r[2, i]
    hi = meta_r[3, i]
    tile = meta_r[0, i]
    rows = tile * BM + lax.broadcasted_iota(jnp.int32, (BM, 1), 0)
    mask = (rows >= lo) & (rows < hi)
    wu = lax.bitcast_convert_type(xs_ref[...], jnp.uint32)
    lo_bf = lax.bitcast_convert_type(
        lax.convert_element_type(
            jnp.bitwise_and(wu, jnp.uint32(0xFFFF)), jnp.uint16),
        jnp.bfloat16)
    hi_bf = lax.bitcast_convert_type(
        lax.convert_element_type(
            lax.shift_right_logical(wu, jnp.uint32(16)), jnp.uint16),
        jnp.bfloat16)
    xb = jnp.where(mask, jnp.concatenate([lo_bf, hi_bf], axis=1),
                   jnp.bfloat16(0))
    dn = (((1,), (1,)), ((), ()))
    z1 = lax.dot_general(xb, w1_ref[0].astype(jnp.bfloat16), dn,
                         preferred_element_type=jnp.float32).astype(jnp.bfloat16)
    z3 = lax.dot_general(xb, w3_ref[0].astype(jnp.bfloat16), dn,
                         preferred_element_type=jnp.float32).astype(jnp.bfloat16)
    h = (z1 * jax.nn.sigmoid(z1)) * z3
    o = lax.dot_general(h, w2_ref[0].astype(jnp.bfloat16), dn,
                        preferred_element_type=jnp.float32)
    o = o.astype(jnp.bfloat16).astype(jnp.float32)

    @pl.when(meta_r[4, i] == 1)
    def _():
        out_ref[...] = o

    @pl.when(meta_r[4, i] == 0)
    def _():
        out_ref[...] = out_ref[...] + o


def _gmm(xs, w1b, w3b, w2b, meta):
    grid_spec = pltpu.PrefetchScalarGridSpec(
        num_scalar_prefetch=1,
        grid=(MAX_VISITS,),
        in_specs=[
            pl.BlockSpec((BM, DIMH), lambda i, m: (m[0, i], 0)),
            pl.BlockSpec((1, DIM, DIM), lambda i, m: (m[1, i], 0, 0)),
            pl.BlockSpec((1, DIM, DIM), lambda i, m: (m[1, i], 0, 0)),
            pl.BlockSpec((1, DIM, DIM), lambda i, m: (m[1, i], 0, 0)),
        ],
        out_specs=pl.BlockSpec((BM, DIM), lambda i, m: (m[0, i], 0)),
    )
    return pl.pallas_call(
        _gmm_body,
        grid_spec=grid_spec,
        out_shape=jax.ShapeDtypeStruct((N, DIM), jnp.float32),
        compiler_params=pltpu.CompilerParams(
            dimension_semantics=("arbitrary",)),
    )(meta, xs, w1b, w3b, w2b)


def _combine_body(y_hbm, dest_hbm, out_hbm, idx_v, rows_v, sem):
    wid = lax.axis_index("s") * 2 + lax.axis_index("c")
    base = wid * ROWS_W
    pltpu.sync_copy(dest_hbm.at[pl.ds(base, ROWS_W)], idx_v)
    pltpu.async_copy(y_hbm.at[idx_v], rows_v, sem).wait()
    pltpu.sync_copy(rows_v, out_hbm.at[pl.ds(base, ROWS_W)])


def _combine(yw, dest):
    mesh = plsc.VectorSubcoreMesh(core_axis_name="c", subcore_axis_name="s")
    return pl.kernel(
        _combine_body,
        out_type=jax.ShapeDtypeStruct((N, DIM), jnp.float32),
        mesh=mesh,
        scratch_types=[
            pltpu.VMEM((ROWS_W,), jnp.int32),
            pltpu.VMEM((ROWS_W, DIM), jnp.float32),
            pltpu.SemaphoreType.DMA,
        ],
    )(yw, dest)


@jax.jit
def kernel(x, gate_w, w1, w2, w3):
    bs, slen, dim = x.shape
    x2d = x.reshape(N, DIM)

    dest2d, meta, xsc = _router(x2d, gate_w)
    dest = dest2d.reshape(N)

    xs = _dispatch(xsc, dest)
    y = _gmm(xs, w1, w3, w2, meta)

    out = _combine(y, dest)
    return out.reshape(bs, slen, dim)
